# Initial kernel scaffold; baseline (speedup 1.0000x reference)
import jax
import jax.numpy as jnp
from jax.experimental import pallas as pl


def kernel(features, cluster_features, edge_index, raw_affine, W_self0, W_neigh0, b0, W_self1, W_neigh1, b1, W_self2, W_neigh2, b2, W_self3, W_neigh3, b3, W_src, b_src, W_dst, b_dst, a1, W_c1, b_c1, a2, W_c2, b_c2):
    raise NotImplementedError("write your pallas kernel here")


# speedup vs baseline: 3.5444x; 3.5444x over previous
import functools

import jax
import jax.numpy as jnp
from jax import lax
from jax.experimental import pallas as pl
from jax.experimental.pallas import tpu as pltpu
from jax.experimental.pallas import tpu_sc as plsc

N = 10000
NPAD = 10240
E = 320000
NC, NS = 2, 16
NW = NC * NS
EPT = E // NW
CK = 80
NCHUNK = EPT // CK
RPS = NPAD // NS
RPW = NPAD // NW

BN = 1024
GN = NPAD // BN

_mesh = plsc.VectorSubcoreMesh(
    core_axis_name="c", subcore_axis_name="s", num_cores=NC, num_subcores=NS)


def _seg_sum_body(with_deg, *refs):
    if with_deg:
        (h_hbm, src_hbm, dst_hbm, z2_hbm, z1_hbm, ones_hbm,
         out_hbm, dout_hbm,
         acc, dacc, sidx, didx, rows, onesv, sem) = refs
    else:
        (h_hbm, src_hbm, dst_hbm, z2_hbm,
         out_hbm,
         acc, sidx, didx, rows, sem) = refs
    c = lax.axis_index("c")
    s = lax.axis_index("s")
    wid = c * NS + s

    r0 = s * RPS
    pltpu.sync_copy(z2_hbm.at[pl.ds(r0, RPS)], acc.at[pl.ds(r0, RPS)])
    if with_deg:
        pltpu.sync_copy(z1_hbm.at[pl.ds(r0, RPS)], dacc.at[pl.ds(r0, RPS)])
        pltpu.sync_copy(ones_hbm, onesv)
    plsc.subcore_barrier()

    e0 = wid * EPT

    def step(i, carry):
        base = e0 + i * CK
        pltpu.sync_copy(src_hbm.at[pl.ds(base, CK)], sidx)
        pltpu.sync_copy(dst_hbm.at[pl.ds(base, CK)], didx)
        pltpu.async_copy(h_hbm.at[sidx], rows, sem).wait()
        pltpu.sync_copy(rows, acc.at[didx], add=True)
        if with_deg:
            pltpu.sync_copy(onesv, dacc.at[didx], add=True)
        return carry

    lax.fori_loop(0, NCHUNK, step, 0)
    plsc.subcore_barrier()

    pltpu.sync_copy(acc.at[pl.ds(r0, RPS)], out_hbm.at[c, pl.ds(r0, RPS)])
    if with_deg:
        pltpu.sync_copy(dacc.at[pl.ds(r0, RPS)],
                        dout_hbm.at[pl.ds(c * NPAD + r0, RPS)])


def _make_seg_sum(with_deg):
    out_type = [jax.ShapeDtypeStruct((NC, NPAD, 128), jnp.float32)]
    scratch = [
        pltpu.VMEM_SHARED((NPAD, 128), jnp.float32),
        pltpu.VMEM((CK,), jnp.int32),
        pltpu.VMEM((CK,), jnp.int32),
        pltpu.VMEM((CK, 128), jnp.float32),
        pltpu.SemaphoreType.DMA,
    ]
    if with_deg:
        out_type.append(jax.ShapeDtypeStruct((NC * NPAD,), jnp.float32))
        scratch.insert(1, pltpu.VMEM_SHARED((NPAD,), jnp.float32))
        scratch.insert(5, pltpu.VMEM((CK,), jnp.float32))
    return pl.kernel(
        functools.partial(_seg_sum_body, with_deg),
        out_type=out_type, mesh=_mesh, scratch_types=scratch,
        name="sc_seg_sum%s" % ("_deg" if with_deg else ""))


_seg_sum_deg = _make_seg_sum(True)
_seg_sum = _make_seg_sum(False)


def _rdeg_body(degp_hbm, out_hbm, a_v, b_v):
    c = lax.axis_index("c")
    s = lax.axis_index("s")
    wid = c * NS + s
    r0 = wid * RPW
    pltpu.sync_copy(degp_hbm.at[pl.ds(r0, RPW)], a_v)
    pltpu.sync_copy(degp_hbm.at[pl.ds(NPAD + r0, RPW)], b_v)

    def step(j, carry):
        o = j * 16
        v = a_v[pl.ds(o, 16)] + b_v[pl.ds(o, 16)]
        a_v[pl.ds(o, 16)] = 1.0 / jnp.maximum(v, 1.0)
        return carry

    lax.fori_loop(0, RPW // 16, step, 0)
    pltpu.sync_copy(a_v, out_hbm.at[pl.ds(r0, RPW)])


_rdeg_kernel = pl.kernel(
    _rdeg_body,
    out_type=[jax.ShapeDtypeStruct((NPAD,), jnp.float32)],
    mesh=_mesh,
    scratch_types=[pltpu.VMEM((RPW,), jnp.float32),
                   pltpu.VMEM((RPW,), jnp.float32)],
    name="sc_rdeg")


def _edge_gather_body(hsd_hbm, src_hbm, dst_hbm, gs_hbm, gd_hbm,
                      sidx, didx, rows_s, rows_d, sem_s, sem_d):
    c = lax.axis_index("c")
    s = lax.axis_index("s")
    wid = c * NS + s
    e0 = wid * EPT

    def step(i, carry):
        base = e0 + i * CK
        pltpu.sync_copy(src_hbm.at[pl.ds(base, CK)], sidx)
        pltpu.sync_copy(dst_hbm.at[pl.ds(base, CK)], didx)
        a = pltpu.async_copy(hsd_hbm.at[sidx], rows_s, sem_s)
        b = pltpu.async_copy(hsd_hbm.at[didx], rows_d, sem_d)
        a.wait()
        b.wait()
        pltpu.sync_copy(rows_s, gs_hbm.at[pl.ds(base, CK)])
        pltpu.sync_copy(rows_d, gd_hbm.at[pl.ds(base, CK)])
        return carry

    lax.fori_loop(0, NCHUNK, step, 0)


_edge_gather = pl.kernel(
    _edge_gather_body,
    out_type=[jax.ShapeDtypeStruct((E, 128), jnp.float32),
              jax.ShapeDtypeStruct((E, 128), jnp.float32)],
    mesh=_mesh,
    scratch_types=[pltpu.VMEM((CK,), jnp.int32),
                   pltpu.VMEM((CK,), jnp.int32),
                   pltpu.VMEM((CK, 128), jnp.float32),
                   pltpu.VMEM((CK, 128), jnp.float32),
                   pltpu.SemaphoreType.DMA,
                   pltpu.SemaphoreType.DMA],
    name="sc_edge_gather")


def _msg_scatter_body(msg_hbm, dst_hbm, z1_hbm, out_hbm,
                      acc, didx, mval, sem):
    c = lax.axis_index("c")
    s = lax.axis_index("s")
    wid = c * NS + s
    r0 = s * RPS
    pltpu.sync_copy(z1_hbm.at[pl.ds(r0, RPS)], acc.at[pl.ds(r0, RPS)])
    plsc.subcore_barrier()

    e0 = wid * EPT

    def step(i, carry):
        base = e0 + i * CK
        pltpu.sync_copy(dst_hbm.at[pl.ds(base, CK)], didx)
        pltpu.sync_copy(msg_hbm.at[pl.ds(base, CK)], mval)
        pltpu.sync_copy(mval, acc.at[didx], add=True)
        return carry

    lax.fori_loop(0, NCHUNK, step, 0)
    plsc.subcore_barrier()
    pltpu.sync_copy(acc.at[pl.ds(r0, RPS)], out_hbm.at[pl.ds(c * NPAD + r0, RPS)])


_msg_scatter = pl.kernel(
    _msg_scatter_body,
    out_type=[jax.ShapeDtypeStruct((NC * NPAD,), jnp.float32)],
    mesh=_mesh,
    scratch_types=[pltpu.VMEM_SHARED((NPAD,), jnp.float32),
                   pltpu.VMEM((CK,), jnp.int32),
                   pltpu.VMEM((CK,), jnp.float32),
                   pltpu.SemaphoreType.DMA],
    name="sc_msg_scatter")


def _pred_den_body(mp_hbm, rdeg_hbm, out_hbm, a_v, b_v, r_v):
    c = lax.axis_index("c")
    s = lax.axis_index("s")
    wid = c * NS + s
    r0 = wid * RPW
    pltpu.sync_copy(mp_hbm.at[pl.ds(r0, RPW)], a_v)
    pltpu.sync_copy(mp_hbm.at[pl.ds(NPAD + r0, RPW)], b_v)
    pltpu.sync_copy(rdeg_hbm.at[pl.ds(r0, RPW)], r_v)

    def step(j, carry):
        o = j * 16
        a_v[pl.ds(o, 16)] = (a_v[pl.ds(o, 16)] + b_v[pl.ds(o, 16)]) * r_v[pl.ds(o, 16)]
        return carry

    lax.fori_loop(0, RPW // 16, step, 0)
    pltpu.sync_copy(a_v, out_hbm.at[pl.ds(r0, RPW)])


_pred_den_kernel = pl.kernel(
    _pred_den_body,
    out_type=[jax.ShapeDtypeStruct((NPAD,), jnp.float32)],
    mesh=_mesh,
    scratch_types=[pltpu.VMEM((RPW,), jnp.float32),
                   pltpu.VMEM((RPW,), jnp.float32),
                   pltpu.VMEM((RPW,), jnp.float32)],
    name="sc_pred_den")


def _dot(a, b):
    return jnp.dot(a, b, preferred_element_type=jnp.float32)


def _tc0_body(f_ref, cf_ref, wn_ref, ws_ref, b_ref, h_ref, s_ref):
    f = f_ref[...]
    cf = cf_ref[...]
    h_ref[...] = _dot(f, wn_ref[0:128, :]) + _dot(cf, wn_ref[128:256, :])
    s_ref[...] = (_dot(f, ws_ref[0:128, :]) + _dot(cf, ws_ref[128:256, :])
                  + b_ref[...])


def _tc0(f, cf, wn, ws, b):
    return pl.pallas_call(
        _tc0_body,
        grid=(GN,),
        in_specs=[
            pl.BlockSpec((BN, 128), lambda i: (i, 0)),
            pl.BlockSpec((BN, 128), lambda i: (i, 0)),
            pl.BlockSpec((256, 128), lambda i: (0, 0)),
            pl.BlockSpec((256, 128), lambda i: (0, 0)),
            pl.BlockSpec((1, 128), lambda i: (0, 0)),
        ],
        out_specs=[
            pl.BlockSpec((BN, 128), lambda i: (i, 0)),
            pl.BlockSpec((BN, 128), lambda i: (i, 0)),
        ],
        out_shape=[jax.ShapeDtypeStruct((NPAD, 128), jnp.float32),
                   jax.ShapeDtypeStruct((NPAD, 128), jnp.float32)],
    )(f, cf, wn, ws, b)


def _tc_layer_body(din, do, p_ref, sp_ref, rd_ref, wn_ref, ws_ref, b_ref,
                   h_ref, s_ref):
    agg = (p_ref[0, :, 0:din] + p_ref[1, :, 0:din]) * rd_ref[...]
    x = jnp.maximum(sp_ref[...] + agg, 0.0)
    h = _dot(x, wn_ref[...])
    if do < 128:
        h_ref[:, 0:do] = h
        h_ref[:, do:128] = jnp.zeros((x.shape[0], 128 - do), jnp.float32)
    else:
        h_ref[...] = h
    s_ref[...] = _dot(x, ws_ref[...]) + b_ref[...]


def _tc_layer(p, s_prev, rdeg, wn, ws, b, din, do):
    return pl.pallas_call(
        functools.partial(_tc_layer_body, din, do),
        grid=(GN,),
        in_specs=[
            pl.BlockSpec((NC, BN, 128), lambda i: (0, i, 0)),
            pl.BlockSpec((BN, din), lambda i: (i, 0)),
            pl.BlockSpec((BN, 1), lambda i: (i, 0)),
            pl.BlockSpec((din, do), lambda i: (0, 0)),
            pl.BlockSpec((din, do), lambda i: (0, 0)),
            pl.BlockSpec((1, do), lambda i: (0, 0)),
        ],
        out_specs=[
            pl.BlockSpec((BN, 128), lambda i: (i, 0)),
            pl.BlockSpec((BN, do), lambda i: (i, 0)),
        ],
        out_shape=[jax.ShapeDtypeStruct((NPAD, 128), jnp.float32),
                   jax.ShapeDtypeStruct((NPAD, do), jnp.float32)],
    )(p, s_prev, rdeg, wn, ws, b)


def _tc_head_body(p_ref, sp_ref, rd_ref, wsrc_ref, wdst_ref, b_ref, o_ref):
    agg = (p_ref[0, :, 0:64] + p_ref[1, :, 0:64]) * rd_ref[...]
    x = jnp.maximum(sp_ref[...] + agg, 0.0)
    o_ref[:, 0:64] = _dot(x, wsrc_ref[...])
    o_ref[:, 64:128] = _dot(x, wdst_ref[...]) + b_ref[...]


def _tc_head(p, s_prev, rdeg, wsrc, wdst, bsum):
    return pl.pallas_call(
        _tc_head_body,
        grid=(GN,),
        in_specs=[
            pl.BlockSpec((NC, BN, 128), lambda i: (0, i, 0)),
            pl.BlockSpec((BN, 64), lambda i: (i, 0)),
            pl.BlockSpec((BN, 1), lambda i: (i, 0)),
            pl.BlockSpec((64, 64), lambda i: (0, 0)),
            pl.BlockSpec((64, 64), lambda i: (0, 0)),
            pl.BlockSpec((1, 64), lambda i: (0, 0)),
        ],
        out_specs=[pl.BlockSpec((BN, 128), lambda i: (i, 0))],
        out_shape=[jax.ShapeDtypeStruct((NPAD, 128), jnp.float32)],
    )(p, s_prev, rdeg, wsrc, wdst, bsum)


BE = 5000
GE = E // BE


def _tc_edge_body(gs_ref, gd_ref, ra_ref, a1_ref, wc1_ref, bc1_ref, a2_ref,
                  wc2_ref, bc2_ref, pc_ref, msg_ref):
    z = gs_ref[:, 0:64] + gd_ref[:, 64:128]
    z = jnp.where(z > 0, z, a1_ref[...] * z)
    y = _dot(z, wc1_ref[...]) + bc1_ref[...]
    y = jnp.where(y > 0, y, a2_ref[...] * y)
    pc = _dot(y, wc2_ref[...]) + bc2_ref[...]
    pc_ref[...] = pc
    d = jnp.tanh((pc[:, 1:2] - pc[:, 0:1]) * 0.5)
    msg_ref[...] = ra_ref[...] * d


def _tc_edge(gs, gd, ra, a1, wc1, bc1, a2, wc2, bc2):
    return pl.pallas_call(
        _tc_edge_body,
        grid=(GE,),
        in_specs=[
            pl.BlockSpec((BE, 128), lambda i: (i, 0)),
            pl.BlockSpec((BE, 128), lambda i: (i, 0)),
            pl.BlockSpec((BE, 1), lambda i: (i, 0)),
            pl.BlockSpec((1, 64), lambda i: (0, 0)),
            pl.BlockSpec((64, 64), lambda i: (0, 0)),
            pl.BlockSpec((1, 64), lambda i: (0, 0)),
            pl.BlockSpec((1, 64), lambda i: (0, 0)),
            pl.BlockSpec((64, 2), lambda i: (0, 0)),
            pl.BlockSpec((1, 2), lambda i: (0, 0)),
        ],
        out_specs=[
            pl.BlockSpec((BE, 2), lambda i: (i, 0)),
            pl.BlockSpec((BE, 1), lambda i: (i, 0)),
        ],
        out_shape=[jax.ShapeDtypeStruct((E, 2), jnp.float32),
                   jax.ShapeDtypeStruct((E, 1), jnp.float32)],
    )(gs, gd, ra, a1, wc1, bc1, a2, wc2, bc2)


@jax.jit
def kernel(features, cluster_features, edge_index, raw_affine,
           W_self0, W_neigh0, b0, W_self1, W_neigh1, b1,
           W_self2, W_neigh2, b2, W_self3, W_neigh3, b3,
           W_src, b_src, W_dst, b_dst, a1, W_c1, b_c1, a2, W_c2, b_c2):
    pad = ((0, NPAD - N), (0, 0))
    f = jnp.pad(features, pad)
    cf = jnp.pad(cluster_features, pad)
    src = edge_index[0].astype(jnp.int32)
    dst = edge_index[1].astype(jnp.int32)

    z128 = jnp.zeros((NPAD, 128), jnp.float32)
    z1 = jnp.zeros((NPAD,), jnp.float32)
    ones = jnp.ones((CK,), jnp.float32)

    h0, s0 = _tc0(f, cf, W_neigh0, W_self0, b0.reshape(1, -1))
    p0, degp = _seg_sum_deg(h0, src, dst, z128, z1, ones)
    (rdeg,) = _rdeg_kernel(degp)
    rdeg_c = rdeg.reshape(NPAD, 1)

    h1, s1 = _tc_layer(p0, s0, rdeg_c, W_neigh1, W_self1, b1.reshape(1, -1),
                       128, 128)
    (p1,) = _seg_sum(h1, src, dst, z128)

    h2, s2 = _tc_layer(p1, s1, rdeg_c, W_neigh2, W_self2, b2.reshape(1, -1),
                       128, 64)
    (p2,) = _seg_sum(h2, src, dst, z128)

    h3, s3 = _tc_layer(p2, s2, rdeg_c, W_neigh3, W_self3, b3.reshape(1, -1),
                       64, 64)
    (p3,) = _seg_sum(h3, src, dst, z128)

    bsum = (b_src + b_dst).reshape(1, -1)
    (hsd,) = _tc_head(p3, s3, rdeg_c, W_src, W_dst, bsum)

    gs, gd = _edge_gather(hsd, src, dst)
    pred_conn, msg = _tc_edge(gs, gd, raw_affine.reshape(E, 1),
                              a1.reshape(1, -1), W_c1, b_c1.reshape(1, -1),
                              a2.reshape(1, -1), W_c2, b_c2.reshape(1, -1))

    (mp,) = _msg_scatter(msg.reshape(E), dst, z1)
    (pred_den,) = _pred_den_kernel(mp, rdeg)

    return pred_conn, pred_den[:N]


# speedup vs baseline: 5.7490x; 1.6220x over previous
import functools

import jax
import jax.numpy as jnp
from jax import lax
from jax.experimental import pallas as pl
from jax.experimental.pallas import tpu as pltpu
from jax.experimental.pallas import tpu_sc as plsc

N = 10000
NPAD = 10240
E = 320000
NC, NS = 2, 16
NW = NC * NS
EPT = E // NW
CK = 80
NCHUNK = EPT // CK
RPS = NPAD // NS
RPW = NPAD // NW

BN = 1024
GN = NPAD // BN

_mesh = plsc.VectorSubcoreMesh(
    core_axis_name="c", subcore_axis_name="s", num_cores=NC, num_subcores=NS)


def _seg_sum_body(with_deg, *refs):
    if with_deg:
        (h_hbm, src_hbm, dst_hbm, z2_hbm, z1_hbm, ones_hbm,
         out_hbm, dout_hbm,
         acc, dacc, sidx0, sidx1, didx0, didx1, rows0, rows1, onesv,
         sem_g0, sem_g1, sem_i0, sem_i1) = refs
    else:
        (h_hbm, src_hbm, dst_hbm, z2_hbm,
         out_hbm,
         acc, sidx0, sidx1, didx0, didx1, rows0, rows1,
         sem_g0, sem_g1, sem_i0, sem_i1) = refs
        onesv = dacc = dout_hbm = None
    sidx = (sidx0, sidx1)
    didx = (didx0, didx1)
    rows = (rows0, rows1)
    sem_g = (sem_g0, sem_g1)
    sem_i = (sem_i0, sem_i1)
    c = lax.axis_index("c")
    s = lax.axis_index("s")
    wid = c * NS + s

    r0 = s * RPS
    pltpu.sync_copy(z2_hbm.at[pl.ds(r0, RPS)], acc.at[pl.ds(r0, RPS)])
    if with_deg:
        pltpu.sync_copy(z1_hbm.at[pl.ds(r0, RPS)], dacc.at[pl.ds(r0, RPS)])
        pltpu.sync_copy(ones_hbm, onesv)
    plsc.subcore_barrier()

    e0 = wid * EPT
    emax = e0 + EPT - CK

    def wait_idx(b):
        pltpu.make_async_copy(src_hbm.at[pl.ds(0, CK)], sidx[b], sem_i[b]).wait()
        pltpu.make_async_copy(dst_hbm.at[pl.ds(0, CK)], didx[b], sem_i[b]).wait()

    def load_idx(b, base):
        pltpu.async_copy(src_hbm.at[pl.ds(base, CK)], sidx[b], sem_i[b])
        pltpu.async_copy(dst_hbm.at[pl.ds(base, CK)], didx[b], sem_i[b])

    def wait_gather(b):
        pltpu.make_async_copy(h_hbm.at[pl.ds(0, CK)], rows[b], sem_g[b]).wait()

    def consume(b):
        pltpu.sync_copy(rows[b], acc.at[didx[b]], add=True)
        if with_deg:
            pltpu.sync_copy(onesv, dacc.at[didx[b]], add=True)

    pltpu.sync_copy(src_hbm.at[pl.ds(e0, CK)], sidx0)
    pltpu.sync_copy(dst_hbm.at[pl.ds(e0, CK)], didx0)
    pltpu.async_copy(h_hbm.at[sidx0], rows0, sem_g0)
    load_idx(1, e0 + CK)

    def pair(g, carry):
        for b in range(2):
            i = 2 * g + b
            cur, nxt = b, 1 - b
            wait_idx(nxt)
            pltpu.async_copy(h_hbm.at[sidx[nxt]], rows[nxt], sem_g[nxt])
            wait_gather(cur)
            consume(cur)
            base2 = jnp.minimum(e0 + (i + 2) * CK, emax)
            load_idx(cur, base2)
        return carry

    lax.fori_loop(0, (NCHUNK - 1) // 2, pair, 0)
    wait_gather(0)
    consume(0)
    wait_idx(1)
    plsc.subcore_barrier()

    pltpu.sync_copy(acc.at[pl.ds(r0, RPS)], out_hbm.at[c, pl.ds(r0, RPS)])
    if with_deg:
        pltpu.sync_copy(dacc.at[pl.ds(r0, RPS)],
                        dout_hbm.at[pl.ds(c * NPAD + r0, RPS)])


def _make_seg_sum(with_deg):
    out_type = [jax.ShapeDtypeStruct((NC, NPAD, 128), jnp.float32)]
    scratch = [
        pltpu.VMEM_SHARED((NPAD, 128), jnp.float32),
        pltpu.VMEM((CK,), jnp.int32),
        pltpu.VMEM((CK,), jnp.int32),
        pltpu.VMEM((CK,), jnp.int32),
        pltpu.VMEM((CK,), jnp.int32),
        pltpu.VMEM((CK, 128), jnp.float32),
        pltpu.VMEM((CK, 128), jnp.float32),
        pltpu.SemaphoreType.DMA,
        pltpu.SemaphoreType.DMA,
        pltpu.SemaphoreType.DMA,
        pltpu.SemaphoreType.DMA,
    ]
    if with_deg:
        out_type.append(jax.ShapeDtypeStruct((NC * NPAD,), jnp.float32))
        scratch.insert(1, pltpu.VMEM_SHARED((NPAD,), jnp.float32))
        scratch.insert(8, pltpu.VMEM((CK,), jnp.float32))
    return pl.kernel(
        functools.partial(_seg_sum_body, with_deg),
        out_type=out_type, mesh=_mesh, scratch_types=scratch,
        name="sc_seg_sum%s" % ("_deg" if with_deg else ""))


_seg_sum_deg = _make_seg_sum(True)
_seg_sum = _make_seg_sum(False)


def _rdeg_body(degp_hbm, out_hbm, a_v, b_v):
    c = lax.axis_index("c")
    s = lax.axis_index("s")
    wid = c * NS + s
    r0 = wid * RPW
    pltpu.sync_copy(degp_hbm.at[pl.ds(r0, RPW)], a_v)
    pltpu.sync_copy(degp_hbm.at[pl.ds(NPAD + r0, RPW)], b_v)

    def step(j, carry):
        o = j * 16
        v = a_v[pl.ds(o, 16)] + b_v[pl.ds(o, 16)]
        a_v[pl.ds(o, 16)] = 1.0 / jnp.maximum(v, 1.0)
        return carry

    lax.fori_loop(0, RPW // 16, step, 0)
    pltpu.sync_copy(a_v, out_hbm.at[pl.ds(r0, RPW)])


_rdeg_kernel = pl.kernel(
    _rdeg_body,
    out_type=[jax.ShapeDtypeStruct((NPAD,), jnp.float32)],
    mesh=_mesh,
    scratch_types=[pltpu.VMEM((RPW,), jnp.float32),
                   pltpu.VMEM((RPW,), jnp.float32)],
    name="sc_rdeg")


def _edge_gather_body(hsd_hbm, src_hbm, dst_hbm, gs_hbm, gd_hbm,
                      sidx0, sidx1, didx0, didx1,
                      rs0, rs1, rd0, rd1,
                      sg0, sg1, si0, si1, sw0, sw1):
    sidx = (sidx0, sidx1)
    didx = (didx0, didx1)
    rows_s = (rs0, rs1)
    rows_d = (rd0, rd1)
    sem_g = (sg0, sg1)
    sem_i = (si0, si1)
    sem_w = (sw0, sw1)
    c = lax.axis_index("c")
    s = lax.axis_index("s")
    wid = c * NS + s
    e0 = wid * EPT
    emax = e0 + EPT - CK

    def wait_idx(b):
        pltpu.make_async_copy(src_hbm.at[pl.ds(0, CK)], sidx[b], sem_i[b]).wait()
        pltpu.make_async_copy(dst_hbm.at[pl.ds(0, CK)], didx[b], sem_i[b]).wait()

    def load_idx(b, base):
        pltpu.async_copy(src_hbm.at[pl.ds(base, CK)], sidx[b], sem_i[b])
        pltpu.async_copy(dst_hbm.at[pl.ds(base, CK)], didx[b], sem_i[b])

    def start_gathers(b):
        pltpu.async_copy(hsd_hbm.at[sidx[b]], rows_s[b], sem_g[b])
        pltpu.async_copy(hsd_hbm.at[didx[b]], rows_d[b], sem_g[b])

    def wait_gathers(b):
        pltpu.make_async_copy(hsd_hbm.at[pl.ds(0, CK)], rows_s[b], sem_g[b]).wait()
        pltpu.make_async_copy(hsd_hbm.at[pl.ds(0, CK)], rows_d[b], sem_g[b]).wait()

    def start_writes(b, base):
        pltpu.async_copy(rows_s[b], gs_hbm.at[pl.ds(base, CK)], sem_w[b])
        pltpu.async_copy(rows_d[b], gd_hbm.at[pl.ds(base, CK)], sem_w[b])

    def wait_writes(b):
        pltpu.make_async_copy(rows_s[b], gs_hbm.at[pl.ds(0, CK)], sem_w[b]).wait()
        pltpu.make_async_copy(rows_d[b], gd_hbm.at[pl.ds(0, CK)], sem_w[b]).wait()

    pltpu.sync_copy(src_hbm.at[pl.ds(e0, CK)], sidx0)
    pltpu.sync_copy(dst_hbm.at[pl.ds(e0, CK)], didx0)
    start_gathers(0)
    load_idx(1, e0 + CK)

    def pair(g, carry):
        for b in range(2):
            i = 2 * g + b
            cur, nxt = b, 1 - b
            if b == 0:
                @pl.when(g > 0)
                def _():
                    wait_writes(nxt)
            else:
                wait_writes(nxt)
            wait_idx(nxt)
            start_gathers(nxt)
            wait_gathers(cur)
            start_writes(cur, e0 + i * CK)
            base2 = jnp.minimum(e0 + (i + 2) * CK, emax)
            load_idx(cur, base2)
        return carry

    lax.fori_loop(0, (NCHUNK - 1) // 2, pair, 0)
    wait_writes(1)
    wait_gathers(0)
    start_writes(0, emax)
    wait_idx(1)
    wait_writes(0)


_edge_gather = pl.kernel(
    _edge_gather_body,
    out_type=[jax.ShapeDtypeStruct((E, 128), jnp.float32),
              jax.ShapeDtypeStruct((E, 128), jnp.float32)],
    mesh=_mesh,
    scratch_types=[pltpu.VMEM((CK,), jnp.int32),
                   pltpu.VMEM((CK,), jnp.int32),
                   pltpu.VMEM((CK,), jnp.int32),
                   pltpu.VMEM((CK,), jnp.int32),
                   pltpu.VMEM((CK, 128), jnp.float32),
                   pltpu.VMEM((CK, 128), jnp.float32),
                   pltpu.VMEM((CK, 128), jnp.float32),
                   pltpu.VMEM((CK, 128), jnp.float32),
                   pltpu.SemaphoreType.DMA,
                   pltpu.SemaphoreType.DMA,
                   pltpu.SemaphoreType.DMA,
                   pltpu.SemaphoreType.DMA,
                   pltpu.SemaphoreType.DMA,
                   pltpu.SemaphoreType.DMA],
    name="sc_edge_gather")


def _msg_scatter_body(msg_hbm, dst_hbm, z1_hbm, out_hbm,
                      acc, didx0, didx1, mval0, mval1, sem0, sem1):
    c = lax.axis_index("c")
    s = lax.axis_index("s")
    wid = c * NS + s
    r0 = s * RPS
    pltpu.sync_copy(z1_hbm.at[pl.ds(r0, RPS)], acc.at[pl.ds(r0, RPS)])
    plsc.subcore_barrier()

    e0 = wid * EPT

    didx = (didx0, didx1)
    mval = (mval0, mval1)
    sem = (sem0, sem1)

    def load(b, base):
        pltpu.async_copy(dst_hbm.at[pl.ds(base, CK)], didx[b], sem[b])
        pltpu.async_copy(msg_hbm.at[pl.ds(base, CK)], mval[b], sem[b])

    def wait_load(b):
        pltpu.make_async_copy(dst_hbm.at[pl.ds(0, CK)], didx[b], sem[b]).wait()
        pltpu.make_async_copy(msg_hbm.at[pl.ds(0, CK)], mval[b], sem[b]).wait()

    load(0, e0)

    def pair(g, carry):
        for b in range(2):
            i = 2 * g + b
            cur, nxt = b, 1 - b
            load(nxt, e0 + (i + 1) * CK)
            wait_load(cur)
            pltpu.sync_copy(mval[cur], acc.at[didx[cur]], add=True)
        return carry

    lax.fori_loop(0, (NCHUNK - 1) // 2, pair, 0)
    wait_load(0)
    pltpu.sync_copy(mval[0], acc.at[didx[0]], add=True)
    plsc.subcore_barrier()
    pltpu.sync_copy(acc.at[pl.ds(r0, RPS)], out_hbm.at[pl.ds(c * NPAD + r0, RPS)])


_msg_scatter = pl.kernel(
    _msg_scatter_body,
    out_type=[jax.ShapeDtypeStruct((NC * NPAD,), jnp.float32)],
    mesh=_mesh,
    scratch_types=[pltpu.VMEM_SHARED((NPAD,), jnp.float32),
                   pltpu.VMEM((CK,), jnp.int32),
                   pltpu.VMEM((CK,), jnp.int32),
                   pltpu.VMEM((CK,), jnp.float32),
                   pltpu.VMEM((CK,), jnp.float32),
                   pltpu.SemaphoreType.DMA,
                   pltpu.SemaphoreType.DMA],
    name="sc_msg_scatter")


def _pred_den_body(mp_hbm, rdeg_hbm, out_hbm, a_v, b_v, r_v):
    c = lax.axis_index("c")
    s = lax.axis_index("s")
    wid = c * NS + s
    r0 = wid * RPW
    pltpu.sync_copy(mp_hbm.at[pl.ds(r0, RPW)], a_v)
    pltpu.sync_copy(mp_hbm.at[pl.ds(NPAD + r0, RPW)], b_v)
    pltpu.sync_copy(rdeg_hbm.at[pl.ds(r0, RPW)], r_v)

    def step(j, carry):
        o = j * 16
        a_v[pl.ds(o, 16)] = (a_v[pl.ds(o, 16)] + b_v[pl.ds(o, 16)]) * r_v[pl.ds(o, 16)]
        return carry

    lax.fori_loop(0, RPW // 16, step, 0)
    pltpu.sync_copy(a_v, out_hbm.at[pl.ds(r0, RPW)])


_pred_den_kernel = pl.kernel(
    _pred_den_body,
    out_type=[jax.ShapeDtypeStruct((NPAD,), jnp.float32)],
    mesh=_mesh,
    scratch_types=[pltpu.VMEM((RPW,), jnp.float32),
                   pltpu.VMEM((RPW,), jnp.float32),
                   pltpu.VMEM((RPW,), jnp.float32)],
    name="sc_pred_den")


def _dot(a, b):
    return jnp.dot(a, b, preferred_element_type=jnp.float32)


def _tc0_body(f_ref, cf_ref, wn_ref, ws_ref, b_ref, h_ref, s_ref):
    f = f_ref[...]
    cf = cf_ref[...]
    h_ref[...] = _dot(f, wn_ref[0:128, :]) + _dot(cf, wn_ref[128:256, :])
    s_ref[...] = (_dot(f, ws_ref[0:128, :]) + _dot(cf, ws_ref[128:256, :])
                  + b_ref[...])


def _tc0(f, cf, wn, ws, b):
    return pl.pallas_call(
        _tc0_body,
        grid=(GN,),
        in_specs=[
            pl.BlockSpec((BN, 128), lambda i: (i, 0)),
            pl.BlockSpec((BN, 128), lambda i: (i, 0)),
            pl.BlockSpec((256, 128), lambda i: (0, 0)),
            pl.BlockSpec((256, 128), lambda i: (0, 0)),
            pl.BlockSpec((1, 128), lambda i: (0, 0)),
        ],
        out_specs=[
            pl.BlockSpec((BN, 128), lambda i: (i, 0)),
            pl.BlockSpec((BN, 128), lambda i: (i, 0)),
        ],
        out_shape=[jax.ShapeDtypeStruct((NPAD, 128), jnp.float32),
                   jax.ShapeDtypeStruct((NPAD, 128), jnp.float32)],
    )(f, cf, wn, ws, b)


def _tc_layer_body(din, do, p_ref, sp_ref, rd_ref, wn_ref, ws_ref, b_ref,
                   h_ref, s_ref):
    agg = (p_ref[0, :, 0:din] + p_ref[1, :, 0:din]) * rd_ref[...]
    x = jnp.maximum(sp_ref[...] + agg, 0.0)
    h = _dot(x, wn_ref[...])
    if do < 128:
        h_ref[:, 0:do] = h
        h_ref[:, do:128] = jnp.zeros((x.shape[0], 128 - do), jnp.float32)
    else:
        h_ref[...] = h
    s_ref[...] = _dot(x, ws_ref[...]) + b_ref[...]


def _tc_layer(p, s_prev, rdeg, wn, ws, b, din, do):
    return pl.pallas_call(
        functools.partial(_tc_layer_body, din, do),
        grid=(GN,),
        in_specs=[
            pl.BlockSpec((NC, BN, 128), lambda i: (0, i, 0)),
            pl.BlockSpec((BN, din), lambda i: (i, 0)),
            pl.BlockSpec((BN, 1), lambda i: (i, 0)),
            pl.BlockSpec((din, do), lambda i: (0, 0)),
            pl.BlockSpec((din, do), lambda i: (0, 0)),
            pl.BlockSpec((1, do), lambda i: (0, 0)),
        ],
        out_specs=[
            pl.BlockSpec((BN, 128), lambda i: (i, 0)),
            pl.BlockSpec((BN, do), lambda i: (i, 0)),
        ],
        out_shape=[jax.ShapeDtypeStruct((NPAD, 128), jnp.float32),
                   jax.ShapeDtypeStruct((NPAD, do), jnp.float32)],
    )(p, s_prev, rdeg, wn, ws, b)


def _tc_head_body(p_ref, sp_ref, rd_ref, wsrc_ref, wdst_ref, b_ref, o_ref):
    agg = (p_ref[0, :, 0:64] + p_ref[1, :, 0:64]) * rd_ref[...]
    x = jnp.maximum(sp_ref[...] + agg, 0.0)
    o_ref[:, 0:64] = _dot(x, wsrc_ref[...])
    o_ref[:, 64:128] = _dot(x, wdst_ref[...]) + b_ref[...]


def _tc_head(p, s_prev, rdeg, wsrc, wdst, bsum):
    return pl.pallas_call(
        _tc_head_body,
        grid=(GN,),
        in_specs=[
            pl.BlockSpec((NC, BN, 128), lambda i: (0, i, 0)),
            pl.BlockSpec((BN, 64), lambda i: (i, 0)),
            pl.BlockSpec((BN, 1), lambda i: (i, 0)),
            pl.BlockSpec((64, 64), lambda i: (0, 0)),
            pl.BlockSpec((64, 64), lambda i: (0, 0)),
            pl.BlockSpec((1, 64), lambda i: (0, 0)),
        ],
        out_specs=[pl.BlockSpec((BN, 128), lambda i: (i, 0))],
        out_shape=[jax.ShapeDtypeStruct((NPAD, 128), jnp.float32)],
    )(p, s_prev, rdeg, wsrc, wdst, bsum)


BE = 5000
GE = E // BE


def _tc_edge_body(gs_ref, gd_ref, ra_ref, a1_ref, wc1_ref, bc1_ref, a2_ref,
                  wc2_ref, bc2_ref, pc_ref, msg_ref):
    z = gs_ref[:, 0:64] + gd_ref[:, 64:128]
    z = jnp.where(z > 0, z, a1_ref[...] * z)
    y = _dot(z, wc1_ref[...]) + bc1_ref[...]
    y = jnp.where(y > 0, y, a2_ref[...] * y)
    pc = _dot(y, wc2_ref[...]) + bc2_ref[...]
    pc_ref[...] = pc
    d = jnp.tanh((pc[:, 1:2] - pc[:, 0:1]) * 0.5)
    msg_ref[...] = ra_ref[...] * d


def _tc_edge(gs, gd, ra, a1, wc1, bc1, a2, wc2, bc2):
    return pl.pallas_call(
        _tc_edge_body,
        grid=(GE,),
        in_specs=[
            pl.BlockSpec((BE, 128), lambda i: (i, 0)),
            pl.BlockSpec((BE, 128), lambda i: (i, 0)),
            pl.BlockSpec((BE, 1), lambda i: (i, 0)),
            pl.BlockSpec((1, 64), lambda i: (0, 0)),
            pl.BlockSpec((64, 64), lambda i: (0, 0)),
            pl.BlockSpec((1, 64), lambda i: (0, 0)),
            pl.BlockSpec((1, 64), lambda i: (0, 0)),
            pl.BlockSpec((64, 2), lambda i: (0, 0)),
            pl.BlockSpec((1, 2), lambda i: (0, 0)),
        ],
        out_specs=[
            pl.BlockSpec((BE, 2), lambda i: (i, 0)),
            pl.BlockSpec((BE, 1), lambda i: (i, 0)),
        ],
        out_shape=[jax.ShapeDtypeStruct((E, 2), jnp.float32),
                   jax.ShapeDtypeStruct((E, 1), jnp.float32)],
    )(gs, gd, ra, a1, wc1, bc1, a2, wc2, bc2)


@jax.jit
def kernel(features, cluster_features, edge_index, raw_affine,
           W_self0, W_neigh0, b0, W_self1, W_neigh1, b1,
           W_self2, W_neigh2, b2, W_self3, W_neigh3, b3,
           W_src, b_src, W_dst, b_dst, a1, W_c1, b_c1, a2, W_c2, b_c2):
    pad = ((0, NPAD - N), (0, 0))
    f = jnp.pad(features, pad)
    cf = jnp.pad(cluster_features, pad)
    src = edge_index[0].astype(jnp.int32)
    dst = edge_index[1].astype(jnp.int32)

    z128 = jnp.zeros((NPAD, 128), jnp.float32)
    z1 = jnp.zeros((NPAD,), jnp.float32)
    ones = jnp.ones((CK,), jnp.float32)

    h0, s0 = _tc0(f, cf, W_neigh0, W_self0, b0.reshape(1, -1))
    p0, degp = _seg_sum_deg(h0, src, dst, z128, z1, ones)
    (rdeg,) = _rdeg_kernel(degp)
    rdeg_c = rdeg.reshape(NPAD, 1)

    h1, s1 = _tc_layer(p0, s0, rdeg_c, W_neigh1, W_self1, b1.reshape(1, -1),
                       128, 128)
    (p1,) = _seg_sum(h1, src, dst, z128)

    h2, s2 = _tc_layer(p1, s1, rdeg_c, W_neigh2, W_self2, b2.reshape(1, -1),
                       128, 64)
    (p2,) = _seg_sum(h2, src, dst, z128)

    h3, s3 = _tc_layer(p2, s2, rdeg_c, W_neigh3, W_self3, b3.reshape(1, -1),
                       64, 64)
    (p3,) = _seg_sum(h3, src, dst, z128)

    bsum = (b_src + b_dst).reshape(1, -1)
    (hsd,) = _tc_head(p3, s3, rdeg_c, W_src, W_dst, bsum)

    gs, gd = _edge_gather(hsd, src, dst)
    pred_conn, msg = _tc_edge(gs, gd, raw_affine.reshape(E, 1),
                              a1.reshape(1, -1), W_c1, b_c1.reshape(1, -1),
                              a2.reshape(1, -1), W_c2, b_c2.reshape(1, -1))

    (mp,) = _msg_scatter(msg.reshape(E), dst, z1)
    (pred_den,) = _pred_den_kernel(mp, rdeg)

    return pred_conn, pred_den[:N]


# speedup vs baseline: 6.2072x; 1.0797x over previous
import functools

import jax
import jax.numpy as jnp
from jax import lax
from jax.experimental import pallas as pl
from jax.experimental.pallas import tpu as pltpu
from jax.experimental.pallas import tpu_sc as plsc

N = 10000
NPAD = 10240
E = 320000
NC, NS = 2, 16
NW = NC * NS
EPT = E // NW
CK = 80
NCHUNK = EPT // CK
RPS = NPAD // NS
RPW = NPAD // NW

BN = 1024
GN = NPAD // BN

_mesh = plsc.VectorSubcoreMesh(
    core_axis_name="c", subcore_axis_name="s", num_cores=NC, num_subcores=NS)


def _seg_sum_body(with_deg, *refs):
    if with_deg:
        (h_hbm, src_hbm, dst_hbm, z2_hbm, z1_hbm, ones_hbm,
         out_hbm, dout_hbm,
         acc, dacc, sidx0, sidx1, didx0, didx1, rows0, rows1, onesv,
         sem_g0, sem_g1, sem_i0, sem_i1) = refs
    else:
        (h_hbm, src_hbm, dst_hbm, z2_hbm,
         out_hbm,
         acc, sidx0, sidx1, didx0, didx1, rows0, rows1,
         sem_g0, sem_g1, sem_i0, sem_i1) = refs
        onesv = dacc = dout_hbm = None
    sidx = (sidx0, sidx1)
    didx = (didx0, didx1)
    rows = (rows0, rows1)
    sem_g = (sem_g0, sem_g1)
    sem_i = (sem_i0, sem_i1)
    c = lax.axis_index("c")
    s = lax.axis_index("s")
    wid = c * NS + s

    r0 = s * RPS
    pltpu.sync_copy(z2_hbm.at[pl.ds(r0, RPS)], acc.at[pl.ds(r0, RPS)])
    if with_deg:
        pltpu.sync_copy(z1_hbm.at[pl.ds(r0, RPS)], dacc.at[pl.ds(r0, RPS)])
        pltpu.sync_copy(ones_hbm, onesv)
    plsc.subcore_barrier()

    e0 = wid * EPT
    emax = e0 + EPT - CK

    def wait_idx(b):
        pltpu.make_async_copy(src_hbm.at[pl.ds(0, CK)], sidx[b], sem_i[b]).wait()
        pltpu.make_async_copy(dst_hbm.at[pl.ds(0, CK)], didx[b], sem_i[b]).wait()

    def load_idx(b, base):
        pltpu.async_copy(src_hbm.at[pl.ds(base, CK)], sidx[b], sem_i[b])
        pltpu.async_copy(dst_hbm.at[pl.ds(base, CK)], didx[b], sem_i[b])

    def wait_gather(b):
        pltpu.make_async_copy(h_hbm.at[pl.ds(0, CK)], rows[b], sem_g[b]).wait()

    def consume(b):
        pltpu.sync_copy(rows[b], acc.at[didx[b]], add=True)
        if with_deg:
            pltpu.sync_copy(onesv, dacc.at[didx[b]], add=True)

    pltpu.sync_copy(src_hbm.at[pl.ds(e0, CK)], sidx0)
    pltpu.sync_copy(dst_hbm.at[pl.ds(e0, CK)], didx0)
    pltpu.async_copy(h_hbm.at[sidx0], rows0, sem_g0)
    load_idx(1, e0 + CK)

    def pair(g, carry):
        for b in range(2):
            i = 2 * g + b
            cur, nxt = b, 1 - b
            wait_idx(nxt)
            pltpu.async_copy(h_hbm.at[sidx[nxt]], rows[nxt], sem_g[nxt])
            wait_gather(cur)
            consume(cur)
            base2 = jnp.minimum(e0 + (i + 2) * CK, emax)
            load_idx(cur, base2)
        return carry

    lax.fori_loop(0, (NCHUNK - 1) // 2, pair, 0)
    wait_gather(0)
    consume(0)
    wait_idx(1)
    plsc.subcore_barrier()

    pltpu.sync_copy(acc.at[pl.ds(r0, RPS)], out_hbm.at[c, pl.ds(r0, RPS)])
    if with_deg:
        pltpu.sync_copy(dacc.at[pl.ds(r0, RPS)],
                        dout_hbm.at[pl.ds(c * NPAD + r0, RPS)])


def _make_seg_sum(with_deg):
    out_type = [jax.ShapeDtypeStruct((NC, NPAD, 128), jnp.float32)]
    scratch = [
        pltpu.VMEM_SHARED((NPAD, 128), jnp.float32),
        pltpu.VMEM((CK,), jnp.int32),
        pltpu.VMEM((CK,), jnp.int32),
        pltpu.VMEM((CK,), jnp.int32),
        pltpu.VMEM((CK,), jnp.int32),
        pltpu.VMEM((CK, 128), jnp.float32),
        pltpu.VMEM((CK, 128), jnp.float32),
        pltpu.SemaphoreType.DMA,
        pltpu.SemaphoreType.DMA,
        pltpu.SemaphoreType.DMA,
        pltpu.SemaphoreType.DMA,
    ]
    if with_deg:
        out_type.append(jax.ShapeDtypeStruct((NC * NPAD,), jnp.float32))
        scratch.insert(1, pltpu.VMEM_SHARED((NPAD,), jnp.float32))
        scratch.insert(8, pltpu.VMEM((CK,), jnp.float32))
    return pl.kernel(
        functools.partial(_seg_sum_body, with_deg),
        out_type=out_type, mesh=_mesh, scratch_types=scratch,
        name="sc_seg_sum%s" % ("_deg" if with_deg else ""))


_seg_sum_deg = _make_seg_sum(True)
_seg_sum = _make_seg_sum(False)


def _rdeg_body(degp_hbm, out_hbm, a_v, b_v):
    c = lax.axis_index("c")
    s = lax.axis_index("s")
    wid = c * NS + s
    r0 = wid * RPW
    pltpu.sync_copy(degp_hbm.at[pl.ds(r0, RPW)], a_v)
    pltpu.sync_copy(degp_hbm.at[pl.ds(NPAD + r0, RPW)], b_v)

    def step(j, carry):
        o = j * 16
        v = a_v[pl.ds(o, 16)] + b_v[pl.ds(o, 16)]
        a_v[pl.ds(o, 16)] = 1.0 / jnp.maximum(v, 1.0)
        return carry

    lax.fori_loop(0, RPW // 16, step, 0)
    pltpu.sync_copy(a_v, out_hbm.at[pl.ds(r0, RPW)])


_rdeg_kernel = pl.kernel(
    _rdeg_body,
    out_type=[jax.ShapeDtypeStruct((NPAD,), jnp.float32)],
    mesh=_mesh,
    scratch_types=[pltpu.VMEM((RPW,), jnp.float32),
                   pltpu.VMEM((RPW,), jnp.float32)],
    name="sc_rdeg")


def _edge_gather_body(hsd_hbm, src_hbm, dst_hbm, ze_hbm,
                      sidx0, sidx1, didx0, didx1,
                      rs0, rs1, rd0, rd1, zb0, zb1,
                      sg0, sg1, si0, si1, sw0, sw1):
    sidx = (sidx0, sidx1)
    didx = (didx0, didx1)
    rows_s = (rs0, rs1)
    rows_d = (rd0, rd1)
    zbuf = (zb0, zb1)
    sem_g = (sg0, sg1)
    sem_i = (si0, si1)
    sem_w = (sw0, sw1)
    c = lax.axis_index("c")
    s = lax.axis_index("s")
    wid = c * NS + s
    e0 = wid * EPT
    emax = e0 + EPT - CK

    def wait_idx(b):
        pltpu.make_async_copy(src_hbm.at[pl.ds(0, CK)], sidx[b], sem_i[b]).wait()
        pltpu.make_async_copy(dst_hbm.at[pl.ds(0, CK)], didx[b], sem_i[b]).wait()

    def load_idx(b, base):
        pltpu.async_copy(src_hbm.at[pl.ds(base, CK)], sidx[b], sem_i[b])
        pltpu.async_copy(dst_hbm.at[pl.ds(base, CK)], didx[b], sem_i[b])

    def start_gathers(b):
        pltpu.async_copy(hsd_hbm.at[sidx[b]], rows_s[b], sem_g[b])
        pltpu.async_copy(hsd_hbm.at[didx[b]], rows_d[b], sem_g[b])

    def wait_gathers(b):
        pltpu.make_async_copy(hsd_hbm.at[pl.ds(0, CK)], rows_s[b], sem_g[b]).wait()
        pltpu.make_async_copy(hsd_hbm.at[pl.ds(0, CK)], rows_d[b], sem_g[b]).wait()

    def compute(b):
        def crow(r, carry):
            for k in range(4):
                zbuf[b][r, pl.ds(16 * k, 16)] = (
                    rows_s[b][r, pl.ds(16 * k, 16)]
                    + rows_d[b][r, pl.ds(64 + 16 * k, 16)])
            return carry
        lax.fori_loop(0, CK, crow, 0)

    def start_write(b, base):
        pltpu.async_copy(zbuf[b], ze_hbm.at[pl.ds(base, CK)], sem_w[b])

    def wait_write(b):
        pltpu.make_async_copy(zbuf[b], ze_hbm.at[pl.ds(0, CK)], sem_w[b]).wait()

    pltpu.sync_copy(src_hbm.at[pl.ds(e0, CK)], sidx0)
    pltpu.sync_copy(dst_hbm.at[pl.ds(e0, CK)], didx0)
    start_gathers(0)
    load_idx(1, e0 + CK)

    def pair(g, carry):
        for b in range(2):
            i = 2 * g + b
            cur, nxt = b, 1 - b
            if b == 0:
                @pl.when(g > 0)
                def _():
                    wait_write(nxt)
            else:
                wait_write(nxt)
            wait_idx(nxt)
            start_gathers(nxt)
            wait_gathers(cur)
            compute(cur)
            start_write(cur, e0 + i * CK)
            base2 = jnp.minimum(e0 + (i + 2) * CK, emax)
            load_idx(cur, base2)
        return carry

    lax.fori_loop(0, (NCHUNK - 1) // 2, pair, 0)
    wait_write(1)
    wait_gathers(0)
    compute(0)
    start_write(0, emax)
    wait_idx(1)
    wait_write(0)


_edge_gather = pl.kernel(
    _edge_gather_body,
    out_type=[jax.ShapeDtypeStruct((E, 64), jnp.float32)],
    mesh=_mesh,
    scratch_types=[pltpu.VMEM((CK,), jnp.int32),
                   pltpu.VMEM((CK,), jnp.int32),
                   pltpu.VMEM((CK,), jnp.int32),
                   pltpu.VMEM((CK,), jnp.int32),
                   pltpu.VMEM((CK, 128), jnp.float32),
                   pltpu.VMEM((CK, 128), jnp.float32),
                   pltpu.VMEM((CK, 128), jnp.float32),
                   pltpu.VMEM((CK, 128), jnp.float32),
                   pltpu.VMEM((CK, 64), jnp.float32),
                   pltpu.VMEM((CK, 64), jnp.float32),
                   pltpu.SemaphoreType.DMA,
                   pltpu.SemaphoreType.DMA,
                   pltpu.SemaphoreType.DMA,
                   pltpu.SemaphoreType.DMA,
                   pltpu.SemaphoreType.DMA,
                   pltpu.SemaphoreType.DMA],
    name="sc_edge_gather")


def _msg_scatter_body(msg_hbm, dst_hbm, z1_hbm, out_hbm,
                      acc, didx0, didx1, mval0, mval1, sem0, sem1):
    c = lax.axis_index("c")
    s = lax.axis_index("s")
    wid = c * NS + s
    r0 = s * RPS
    pltpu.sync_copy(z1_hbm.at[pl.ds(r0, RPS)], acc.at[pl.ds(r0, RPS)])
    plsc.subcore_barrier()

    e0 = wid * EPT

    didx = (didx0, didx1)
    mval = (mval0, mval1)
    sem = (sem0, sem1)

    def load(b, base):
        pltpu.async_copy(dst_hbm.at[pl.ds(base, CK)], didx[b], sem[b])
        pltpu.async_copy(msg_hbm.at[pl.ds(base, CK)], mval[b], sem[b])

    def wait_load(b):
        pltpu.make_async_copy(dst_hbm.at[pl.ds(0, CK)], didx[b], sem[b]).wait()
        pltpu.make_async_copy(msg_hbm.at[pl.ds(0, CK)], mval[b], sem[b]).wait()

    load(0, e0)

    def pair(g, carry):
        for b in range(2):
            i = 2 * g + b
            cur, nxt = b, 1 - b
            load(nxt, e0 + (i + 1) * CK)
            wait_load(cur)
            pltpu.sync_copy(mval[cur], acc.at[didx[cur]], add=True)
        return carry

    lax.fori_loop(0, (NCHUNK - 1) // 2, pair, 0)
    wait_load(0)
    pltpu.sync_copy(mval[0], acc.at[didx[0]], add=True)
    plsc.subcore_barrier()
    pltpu.sync_copy(acc.at[pl.ds(r0, RPS)], out_hbm.at[pl.ds(c * NPAD + r0, RPS)])


_msg_scatter = pl.kernel(
    _msg_scatter_body,
    out_type=[jax.ShapeDtypeStruct((NC * NPAD,), jnp.float32)],
    mesh=_mesh,
    scratch_types=[pltpu.VMEM_SHARED((NPAD,), jnp.float32),
                   pltpu.VMEM((CK,), jnp.int32),
                   pltpu.VMEM((CK,), jnp.int32),
                   pltpu.VMEM((CK,), jnp.float32),
                   pltpu.VMEM((CK,), jnp.float32),
                   pltpu.SemaphoreType.DMA,
                   pltpu.SemaphoreType.DMA],
    name="sc_msg_scatter")


def _pred_den_body(mp_hbm, rdeg_hbm, out_hbm, a_v, b_v, r_v):
    c = lax.axis_index("c")
    s = lax.axis_index("s")
    wid = c * NS + s
    r0 = wid * RPW
    pltpu.sync_copy(mp_hbm.at[pl.ds(r0, RPW)], a_v)
    pltpu.sync_copy(mp_hbm.at[pl.ds(NPAD + r0, RPW)], b_v)
    pltpu.sync_copy(rdeg_hbm.at[pl.ds(r0, RPW)], r_v)

    def step(j, carry):
        o = j * 16
        a_v[pl.ds(o, 16)] = (a_v[pl.ds(o, 16)] + b_v[pl.ds(o, 16)]) * r_v[pl.ds(o, 16)]
        return carry

    lax.fori_loop(0, RPW // 16, step, 0)
    pltpu.sync_copy(a_v, out_hbm.at[pl.ds(r0, RPW)])


_pred_den_kernel = pl.kernel(
    _pred_den_body,
    out_type=[jax.ShapeDtypeStruct((NPAD,), jnp.float32)],
    mesh=_mesh,
    scratch_types=[pltpu.VMEM((RPW,), jnp.float32),
                   pltpu.VMEM((RPW,), jnp.float32),
                   pltpu.VMEM((RPW,), jnp.float32)],
    name="sc_pred_den")


def _dot(a, b):
    return jnp.dot(a, b, preferred_element_type=jnp.float32)


def _tc0_body(f_ref, cf_ref, wn_ref, ws_ref, b_ref, h_ref, s_ref):
    f = f_ref[...]
    cf = cf_ref[...]
    h_ref[...] = _dot(f, wn_ref[0:128, :]) + _dot(cf, wn_ref[128:256, :])
    s_ref[...] = (_dot(f, ws_ref[0:128, :]) + _dot(cf, ws_ref[128:256, :])
                  + b_ref[...])


def _tc0(f, cf, wn, ws, b):
    return pl.pallas_call(
        _tc0_body,
        grid=(GN,),
        in_specs=[
            pl.BlockSpec((BN, 128), lambda i: (i, 0)),
            pl.BlockSpec((BN, 128), lambda i: (i, 0)),
            pl.BlockSpec((256, 128), lambda i: (0, 0)),
            pl.BlockSpec((256, 128), lambda i: (0, 0)),
            pl.BlockSpec((1, 128), lambda i: (0, 0)),
        ],
        out_specs=[
            pl.BlockSpec((BN, 128), lambda i: (i, 0)),
            pl.BlockSpec((BN, 128), lambda i: (i, 0)),
        ],
        out_shape=[jax.ShapeDtypeStruct((NPAD, 128), jnp.float32),
                   jax.ShapeDtypeStruct((NPAD, 128), jnp.float32)],
    )(f, cf, wn, ws, b)


def _tc_layer_body(din, do, p_ref, sp_ref, rd_ref, wn_ref, ws_ref, b_ref,
                   h_ref, s_ref):
    agg = (p_ref[0, :, 0:din] + p_ref[1, :, 0:din]) * rd_ref[...]
    x = jnp.maximum(sp_ref[...] + agg, 0.0)
    h = _dot(x, wn_ref[...])
    if do < 128:
        h_ref[:, 0:do] = h
        h_ref[:, do:128] = jnp.zeros((x.shape[0], 128 - do), jnp.float32)
    else:
        h_ref[...] = h
    s_ref[...] = _dot(x, ws_ref[...]) + b_ref[...]


def _tc_layer(p, s_prev, rdeg, wn, ws, b, din, do):
    return pl.pallas_call(
        functools.partial(_tc_layer_body, din, do),
        grid=(GN,),
        in_specs=[
            pl.BlockSpec((NC, BN, 128), lambda i: (0, i, 0)),
            pl.BlockSpec((BN, din), lambda i: (i, 0)),
            pl.BlockSpec((BN, 1), lambda i: (i, 0)),
            pl.BlockSpec((din, do), lambda i: (0, 0)),
            pl.BlockSpec((din, do), lambda i: (0, 0)),
            pl.BlockSpec((1, do), lambda i: (0, 0)),
        ],
        out_specs=[
            pl.BlockSpec((BN, 128), lambda i: (i, 0)),
            pl.BlockSpec((BN, do), lambda i: (i, 0)),
        ],
        out_shape=[jax.ShapeDtypeStruct((NPAD, 128), jnp.float32),
                   jax.ShapeDtypeStruct((NPAD, do), jnp.float32)],
    )(p, s_prev, rdeg, wn, ws, b)


def _tc_head_body(p_ref, sp_ref, rd_ref, wsrc_ref, wdst_ref, b_ref, o_ref):
    agg = (p_ref[0, :, 0:64] + p_ref[1, :, 0:64]) * rd_ref[...]
    x = jnp.maximum(sp_ref[...] + agg, 0.0)
    o_ref[:, 0:64] = _dot(x, wsrc_ref[...])
    o_ref[:, 64:128] = _dot(x, wdst_ref[...]) + b_ref[...]


def _tc_head(p, s_prev, rdeg, wsrc, wdst, bsum):
    return pl.pallas_call(
        _tc_head_body,
        grid=(GN,),
        in_specs=[
            pl.BlockSpec((NC, BN, 128), lambda i: (0, i, 0)),
            pl.BlockSpec((BN, 64), lambda i: (i, 0)),
            pl.BlockSpec((BN, 1), lambda i: (i, 0)),
            pl.BlockSpec((64, 64), lambda i: (0, 0)),
            pl.BlockSpec((64, 64), lambda i: (0, 0)),
            pl.BlockSpec((1, 64), lambda i: (0, 0)),
        ],
        out_specs=[pl.BlockSpec((BN, 128), lambda i: (i, 0))],
        out_shape=[jax.ShapeDtypeStruct((NPAD, 128), jnp.float32)],
    )(p, s_prev, rdeg, wsrc, wdst, bsum)


BE = 5000
GE = E // BE


def _tc_edge_body(ze_ref, ra_ref, a1_ref, wc1_ref, bc1_ref, a2_ref,
                  wc2_ref, bc2_ref, pc_ref, msg_ref):
    z = ze_ref[...]
    z = jnp.where(z > 0, z, a1_ref[...] * z)
    y = _dot(z, wc1_ref[...]) + bc1_ref[...]
    y = jnp.where(y > 0, y, a2_ref[...] * y)
    pc = _dot(y, wc2_ref[...]) + bc2_ref[...]
    pc_ref[...] = pc
    d = jnp.tanh((pc[:, 1:2] - pc[:, 0:1]) * 0.5)
    msg_ref[...] = ra_ref[...] * d


def _tc_edge(ze, ra, a1, wc1, bc1, a2, wc2, bc2):
    return pl.pallas_call(
        _tc_edge_body,
        grid=(GE,),
        in_specs=[
            pl.BlockSpec((BE, 64), lambda i: (i, 0)),
            pl.BlockSpec((BE, 1), lambda i: (i, 0)),
            pl.BlockSpec((1, 64), lambda i: (0, 0)),
            pl.BlockSpec((64, 64), lambda i: (0, 0)),
            pl.BlockSpec((1, 64), lambda i: (0, 0)),
            pl.BlockSpec((1, 64), lambda i: (0, 0)),
            pl.BlockSpec((64, 2), lambda i: (0, 0)),
            pl.BlockSpec((1, 2), lambda i: (0, 0)),
        ],
        out_specs=[
            pl.BlockSpec((BE, 2), lambda i: (i, 0)),
            pl.BlockSpec((BE, 1), lambda i: (i, 0)),
        ],
        out_shape=[jax.ShapeDtypeStruct((E, 2), jnp.float32),
                   jax.ShapeDtypeStruct((E, 1), jnp.float32)],
    )(ze, ra, a1, wc1, bc1, a2, wc2, bc2)


@jax.jit
def kernel(features, cluster_features, edge_index, raw_affine,
           W_self0, W_neigh0, b0, W_self1, W_neigh1, b1,
           W_self2, W_neigh2, b2, W_self3, W_neigh3, b3,
           W_src, b_src, W_dst, b_dst, a1, W_c1, b_c1, a2, W_c2, b_c2):
    pad = ((0, NPAD - N), (0, 0))
    f = jnp.pad(features, pad)
    cf = jnp.pad(cluster_features, pad)
    src = edge_index[0].astype(jnp.int32)
    dst = edge_index[1].astype(jnp.int32)

    z128 = jnp.zeros((NPAD, 128), jnp.float32)
    z1 = jnp.zeros((NPAD,), jnp.float32)
    ones = jnp.ones((CK,), jnp.float32)

    h0, s0 = _tc0(f, cf, W_neigh0, W_self0, b0.reshape(1, -1))
    p0, degp = _seg_sum_deg(h0, src, dst, z128, z1, ones)
    (rdeg,) = _rdeg_kernel(degp)
    rdeg_c = rdeg.reshape(NPAD, 1)

    h1, s1 = _tc_layer(p0, s0, rdeg_c, W_neigh1, W_self1, b1.reshape(1, -1),
                       128, 128)
    (p1,) = _seg_sum(h1, src, dst, z128)

    h2, s2 = _tc_layer(p1, s1, rdeg_c, W_neigh2, W_self2, b2.reshape(1, -1),
                       128, 64)
    (p2,) = _seg_sum(h2, src, dst, z128)

    h3, s3 = _tc_layer(p2, s2, rdeg_c, W_neigh3, W_self3, b3.reshape(1, -1),
                       64, 64)
    (p3,) = _seg_sum(h3, src, dst, z128)

    bsum = (b_src + b_dst).reshape(1, -1)
    (hsd,) = _tc_head(p3, s3, rdeg_c, W_src, W_dst, bsum)

    (ze,) = _edge_gather(hsd, src, dst)
    pred_conn, msg = _tc_edge(ze, raw_affine.reshape(E, 1),
                              a1.reshape(1, -1), W_c1, b_c1.reshape(1, -1),
                              a2.reshape(1, -1), W_c2, b_c2.reshape(1, -1))

    (mp,) = _msg_scatter(msg.reshape(E), dst, z1)
    (pred_den,) = _pred_den_kernel(mp, rdeg)

    return pred_conn, pred_den[:N]
